# Initial kernel scaffold; baseline (speedup 1.0000x reference)
import jax
import jax.numpy as jnp
from jax.experimental import pallas as pl


def kernel(src_feat, pts3D, K):
    raise NotImplementedError("write your pallas kernel here")


# speedup vs baseline: 1.0135x; 1.0135x over previous
import functools

import jax
import jax.numpy as jnp
from jax.experimental import pallas as pl
from jax.experimental.pallas import tpu as pltpu

EPS = 0.01
H, W = 352, 1216
RADIUS_PX = 4.0
WS = float(min(H, W))
RADIUS = RADIUS_PX / float(max(H, W)) * 2.0
TAPS = ((-1, -1), (-1, 0), (-1, 1), (0, -1), (0, 0), (0, 1), (1, -1), (1, 0), (1, 1))

BLK = 8192


def _phase_a_body(sx_ref, sy_ref, sz_ref, i0_ref, j0_ref, w_ref, flat_ref):
    sx = sx_ref[0, 0]
    sy = sy_ref[0, 0]
    sz = sz_ref[0, 0]
    i0 = i0_ref[0, 0]
    j0 = j0_ref[0, 0]
    r2 = RADIUS * RADIUS
    ws_list = []
    flats = []
    for di, dj in TAPS:
        ii = i0 + di
        jj = j0 + dj
        xc = (W - 1.0 - 2.0 * jj.astype(jnp.float32)) / WS
        yc = (H - 1.0 - 2.0 * ii.astype(jnp.float32)) / WS
        d2 = (sx - xc) ** 2 + (sy - yc) ** 2
        inside = ((d2 < r2) & (ii >= 0) & (ii < H) & (jj >= 0) & (jj < W)
                  & (sz > 0.0))
        dist = d2 / r2
        alpha = 1.0 - jnp.sqrt(jnp.clip(dist, 0.001, 1.0))
        wt = jnp.where(inside, alpha, 0.0)
        flat = (jnp.clip(ii, 0, H - 1) * W + jnp.clip(jj, 0, W - 1))
        flat = jnp.where(inside, flat, 0)
        ws_list.append(wt)
        flats.append(flat)
    zf = jnp.zeros_like(ws_list[0])
    zi = jnp.zeros_like(flats[0])
    w_ref[0] = jnp.stack(ws_list + [zf] * 7, axis=0)
    flat_ref[0] = jnp.stack(flats + [zi] * 7, axis=0)


def _project(pts3D, K):
    nK = jnp.zeros_like(K)
    nK = nK.at[:, 0, :].set(K[:, 0, :] / WS)
    nK = nK.at[:, 1, :].set(K[:, 1, :] / WS)
    nK = nK.at[:, 2, 2].set(1.0)
    xy_proj = jnp.einsum('bij,bjn->bin', nK, pts3D)
    mask = jnp.abs(xy_proj[:, 2:3, :]) < EPS
    zs = jnp.where(mask, EPS, xy_proj[:, 2:3, :])
    sampler = jnp.concatenate([
        2.0 * xy_proj[:, 0:1, :] / -zs + W / WS,
        2.0 * xy_proj[:, 1:2, :] / -zs + H / WS,
        xy_proj[:, 2:3, :]], axis=1)
    sampler = jnp.where(jnp.broadcast_to(mask, sampler.shape), -10.0, sampler)
    sx = sampler[:, 0:1, :]
    sy = sampler[:, 1:2, :]
    sz = sampler[:, 2:3, :]
    j0 = jnp.round((W - 1.0 - sx * WS) / 2.0).astype(jnp.int32)
    i0 = jnp.round((H - 1.0 - sy * WS) / 2.0).astype(jnp.int32)
    return sx, sy, sz, i0, j0


@functools.partial(jax.jit, static_argnums=())
def _phase_a(pts3D, K):
    B = pts3D.shape[0]
    N = pts3D.shape[2]
    sx, sy, sz, i0, j0 = _project(pts3D, K)
    grid = (B, N // BLK)
    w, flat = pl.pallas_call(
        _phase_a_body,
        grid=grid,
        in_specs=[
            pl.BlockSpec((1, 1, BLK), lambda b, n: (b, 0, n)),
            pl.BlockSpec((1, 1, BLK), lambda b, n: (b, 0, n)),
            pl.BlockSpec((1, 1, BLK), lambda b, n: (b, 0, n)),
            pl.BlockSpec((1, 1, BLK), lambda b, n: (b, 0, n)),
            pl.BlockSpec((1, 1, BLK), lambda b, n: (b, 0, n)),
        ],
        out_specs=[
            pl.BlockSpec((1, 16, BLK), lambda b, n: (b, 0, n)),
            pl.BlockSpec((1, 16, BLK), lambda b, n: (b, 0, n)),
        ],
        out_shape=[
            jax.ShapeDtypeStruct((B, 16, N), jnp.float32),
            jax.ShapeDtypeStruct((B, 16, N), jnp.int32),
        ],
    )(sx, sy, sz, i0, j0)
    return w, flat


def kernel(src_feat, pts3D, K):
    B, C, N = src_feat.shape
    w, flat = _phase_a(pts3D, K)
    srcT = jnp.transpose(src_feat, (0, 2, 1))
    out = jnp.zeros((B, H * W, C), dtype=src_feat.dtype)

    def scatter_one(o, f, v):
        return o.at[f].add(v)

    for t in range(9):
        vals = srcT * w[:, t, :, None]
        out = jax.vmap(scatter_one)(out, flat[:, t], vals)
    return jnp.transpose(out.reshape(B, H, W, C), (0, 3, 1, 2))


# speedup vs baseline: 1.4415x; 1.4223x over previous
import functools

import jax
import jax.numpy as jnp
from jax import lax
from jax.experimental import pallas as pl
from jax.experimental.pallas import tpu as pltpu
from jax.experimental.pallas import tpu_sc as plsc

EPS = 0.01
H, W = 352, 1216
HW = H * W
RADIUS_PX = 4.0
WS = float(min(H, W))
RADIUS = RADIUS_PX / float(max(H, W)) * 2.0
TAPS = ((-1, -1), (-1, 0), (-1, 1), (0, -1), (0, 0), (0, 1), (1, -1), (1, 0), (1, 1))

BLK = 8192

BAND_ROWS = 16
BAND_PX = BAND_ROWS * W
NBANDS = H // BAND_ROWS
NB_PER_CORE = NBANDS // 2
NSUB = 16
CH = 65536 // NSUB
STRIPE = BAND_PX // NSUB
R = 64
FLUSH_HI = 512
PEND_CAP = FLUSH_HI + CH + 2 * R


def _phase_a_body(sx_ref, sy_ref, sz_ref, i0_ref, j0_ref, w_ref, flat_ref):
    sx = sx_ref[0, 0]
    sy = sy_ref[0, 0]
    sz = sz_ref[0, 0]
    i0 = i0_ref[0, 0]
    j0 = j0_ref[0, 0]
    r2 = RADIUS * RADIUS
    ws_list = []
    flats = []
    for di, dj in TAPS:
        ii = i0 + di
        jj = j0 + dj
        xc = (W - 1.0 - 2.0 * jj.astype(jnp.float32)) / WS
        yc = (H - 1.0 - 2.0 * ii.astype(jnp.float32)) / WS
        d2 = (sx - xc) ** 2 + (sy - yc) ** 2
        inside = ((d2 < r2) & (ii >= 0) & (ii < H) & (jj >= 0) & (jj < W)
                  & (sz > 0.0))
        dist = d2 / r2
        alpha = 1.0 - jnp.sqrt(jnp.clip(dist, 0.001, 1.0))
        wt = jnp.where(inside, alpha, 0.0)
        flat = (jnp.clip(ii, 0, H - 1) * W + jnp.clip(jj, 0, W - 1))
        flat = jnp.where(inside, flat, HW)
        ws_list.append(wt)
        flats.append(flat)
    zf = jnp.zeros_like(ws_list[0])
    si = jnp.full_like(flats[0], HW)
    w_ref[0] = jnp.stack(ws_list + [zf] * 7, axis=0)
    flat_ref[0] = jnp.stack(flats + [si] * 7, axis=0)


def _project(pts3D, K):
    nK = jnp.zeros_like(K)
    nK = nK.at[:, 0, :].set(K[:, 0, :] / WS)
    nK = nK.at[:, 1, :].set(K[:, 1, :] / WS)
    nK = nK.at[:, 2, 2].set(1.0)
    xy_proj = jnp.einsum('bij,bjn->bin', nK, pts3D)
    mask = jnp.abs(xy_proj[:, 2:3, :]) < EPS
    zs = jnp.where(mask, EPS, xy_proj[:, 2:3, :])
    sampler = jnp.concatenate([
        2.0 * xy_proj[:, 0:1, :] / -zs + W / WS,
        2.0 * xy_proj[:, 1:2, :] / -zs + H / WS,
        xy_proj[:, 2:3, :]], axis=1)
    sampler = jnp.where(jnp.broadcast_to(mask, sampler.shape), -10.0, sampler)
    sx = sampler[:, 0:1, :]
    sy = sampler[:, 1:2, :]
    sz = sampler[:, 2:3, :]
    j0 = jnp.round((W - 1.0 - sx * WS) / 2.0).astype(jnp.int32)
    i0 = jnp.round((H - 1.0 - sy * WS) / 2.0).astype(jnp.int32)
    return sx, sy, sz, i0, j0


def _phase_a(pts3D, K):
    B = pts3D.shape[0]
    N = pts3D.shape[2]
    sx, sy, sz, i0, j0 = _project(pts3D, K)
    grid = (B, N // BLK)
    w, flat = pl.pallas_call(
        _phase_a_body,
        grid=grid,
        in_specs=[
            pl.BlockSpec((1, 1, BLK), lambda b, n: (b, 0, n)),
            pl.BlockSpec((1, 1, BLK), lambda b, n: (b, 0, n)),
            pl.BlockSpec((1, 1, BLK), lambda b, n: (b, 0, n)),
            pl.BlockSpec((1, 1, BLK), lambda b, n: (b, 0, n)),
            pl.BlockSpec((1, 1, BLK), lambda b, n: (b, 0, n)),
        ],
        out_specs=[
            pl.BlockSpec((1, 16, BLK), lambda b, n: (b, 0, n)),
            pl.BlockSpec((1, 16, BLK), lambda b, n: (b, 0, n)),
        ],
        out_shape=[
            jax.ShapeDtypeStruct((B, 16, N), jnp.float32),
            jax.ShapeDtypeStruct((B, 16, N), jnp.int32),
        ],
    )(sx, sy, sz, i0, j0)
    return w, flat


def _sc_splat(flat, w, feat2, B, N):
    mesh = plsc.VectorSubcoreMesh(core_axis_name="c", subcore_axis_name="s")

    @functools.partial(
        pl.kernel,
        mesh=mesh,
        out_type=jax.ShapeDtypeStruct((B * HW, 64), jnp.float32),
        compiler_params=pltpu.CompilerParams(
            use_tc_tiling_on_sc=False, needs_layout_passes=False),
        scratch_types=[
            pltpu.VMEM((CH,), jnp.int32),
            pltpu.VMEM((CH,), jnp.float32),
            pltpu.VMEM((PEND_CAP,), jnp.int32),
            pltpu.VMEM((PEND_CAP,), jnp.float32),
            pltpu.VMEM((PEND_CAP,), jnp.int32),
            pltpu.VMEM((R,), jnp.int32),
            pltpu.VMEM((R,), jnp.int32),
            pltpu.VMEM((R,), jnp.float32),
            pltpu.VMEM((R, 64), jnp.float32),
            pltpu.VMEM((R, 64), jnp.float32),
            pltpu.VMEM_SHARED((BAND_PX, 64), jnp.float32),
            pltpu.SemaphoreType.DMA,
        ],
    )
    def k(flat_hbm, w_hbm, feat_hbm, out_hbm, flat_c, w_c, pidx, pw, ppid,
          idx_sub, pid_sub, w_sub, rows, zbuf, acc, sem):
        cid = lax.axis_index("c")
        sid = lax.axis_index("s")
        col0 = sid * CH
        zf16 = jnp.zeros((16,), jnp.float32)
        zi16 = jnp.zeros((16,), jnp.int32)
        lane = jnp.arange(16, dtype=jnp.int32)
        dnums = lax.GatherDimensionNumbers(
            offset_dims=(), collapsed_slice_dims=(0,), start_index_map=(0,))

        def bcast_lane(vec, l):
            idx = jnp.full((16, 1), l, jnp.int32)
            return lax.gather(vec, idx, dimension_numbers=dnums,
                              slice_sizes=(1,),
                              mode=lax.GatherScatterMode.PROMISE_IN_BOUNDS)

        def zb_body(r, _):
            for c4 in range(4):
                zbuf[r, pl.ds(c4 * 16, 16)] = zf16
            return 0
        lax.fori_loop(0, R, zb_body, 0)

        def flush_if(cnt, lo, thresh):
            for kk in range(R // 16):
                pidx[pl.ds(cnt + kk * 16, 16)] = zi16
                pw[pl.ds(cnt + kk * 16, 16)] = zf16
                ppid[pl.ds(cnt + kk * 16, 16)] = zi16
            nb = jnp.where(cnt >= thresh, (cnt + (R - 1)) // R, 0)

            def j_body(j, _):
                off = j * R
                for q in range(R // 16):
                    idx_sub[pl.ds(q * 16, 16)] = pidx[pl.ds(off + q * 16, 16)]
                    pid_sub[pl.ds(q * 16, 16)] = ppid[pl.ds(off + q * 16, 16)]
                    w_sub[pl.ds(q * 16, 16)] = pw[pl.ds(off + q * 16, 16)]
                pltpu.async_copy(feat_hbm.at[pid_sub], rows, sem).wait()

                def q_body(q, _):
                    w16 = w_sub[pl.ds(q * 16, 16)]
                    for l in range(16):
                        wb = bcast_lane(w16, l)
                        ri = q * 16 + l
                        for c4 in range(4):
                            rows[ri, pl.ds(c4 * 16, 16)] = (
                                rows[ri, pl.ds(c4 * 16, 16)] * wb)
                    return 0
                lax.fori_loop(0, R // 16, q_body, 0)
                pltpu.sync_copy(rows, acc.at[idx_sub], add=True)
                return 0
            lax.fori_loop(0, nb, j_body, 0)
            return jnp.where(nb > 0, 0, cnt)

        def band_body(it, _):
            b = it // NB_PER_CORE
            band = (it % NB_PER_CORE) * 2 + cid
            lo = band * BAND_PX
            hi = lo + BAND_PX
            pid_base = b * N + col0

            def z_body(kz, _):
                pltpu.sync_copy(zbuf, acc.at[pl.ds(sid * STRIPE + kz * R, R)])
                return 0
            lax.fori_loop(0, STRIPE // R, z_body, 0)
            plsc.subcore_barrier()

            def row_body(t, cnt):
                pltpu.sync_copy(flat_hbm.at[b, t, pl.ds(col0, CH)], flat_c)
                pltpu.sync_copy(w_hbm.at[b, t, pl.ds(col0, CH)], w_c)

                def g_body(g, cnt):
                    g16 = g * 16
                    fv = flat_c[pl.ds(g16, 16)]
                    wv = w_c[pl.ds(g16, 16)]
                    m = (fv >= lo) & (fv < hi)
                    incl = plsc.cumsum(m.astype(jnp.int32))
                    pos = cnt + incl - 1
                    plsc.store_scatter(pidx, [pos], fv - lo, mask=m)
                    plsc.store_scatter(pw, [pos], wv, mask=m)
                    plsc.store_scatter(ppid, [pos], pid_base + g16 + lane,
                                       mask=m)
                    return cnt + jnp.sum(m.astype(jnp.int32))
                cnt = lax.fori_loop(0, CH // 16, g_body, cnt)
                return flush_if(cnt, lo, FLUSH_HI)

            cnt = lax.fori_loop(0, 9, row_body, jnp.int32(0))
            flush_if(cnt, lo, 1)
            plsc.subcore_barrier()

            def wb_body(kz, _):
                row0 = sid * STRIPE + kz * R
                pltpu.sync_copy(acc.at[pl.ds(row0, R)],
                                out_hbm.at[pl.ds(b * HW + lo + row0, R)])
                return 0
            lax.fori_loop(0, STRIPE // R, wb_body, 0)
            plsc.subcore_barrier()
            return 0

        lax.fori_loop(0, B * NB_PER_CORE, band_body, 0)

    return k(flat, w, feat2)


def kernel(src_feat, pts3D, K):
    B, C, N = src_feat.shape
    w, flat = _phase_a(pts3D, K)
    feat2 = jnp.transpose(src_feat, (0, 2, 1)).reshape(B * N, C)
    out = _sc_splat(flat, w, feat2, B, N)
    return jnp.transpose(out.reshape(B, H, W, C), (0, 3, 1, 2))
